# baseline (speedup 1.0000x reference)
import functools

import jax
import jax.numpy as jnp
from jax import lax
from jax.experimental import pallas as pl
from jax.experimental.pallas import tpu as pltpu
from jax.experimental.pallas import tpu_sc as plsc

D = 32
NC, NS, L = 2, 16, 16
NW = NC * NS
CHUNK = 128
NCH = 4
BPW = NCH * CHUNK
BATCH = NW * BPW

_mesh = plsc.VectorSubcoreMesh(core_axis_name="c", subcore_axis_name="s")


def _gather_body(uidx, pidx, nidx, utab, itab, urows_hbm, prows_hbm, nrows_hbm,
                 uidx_v, pidx_v, nidx_v, urows, prows, nrows,
                 s0, s1, s2, s3):
    sems = (s0, s1, s2, s3)
    wid = lax.axis_index("s") * NC + lax.axis_index("c")
    base = wid * NCH
    pltpu.sync_copy(uidx.at[pl.ds(base, NCH)], uidx_v)
    pltpu.sync_copy(pidx.at[pl.ds(base, NCH)], pidx_v)
    pltpu.sync_copy(nidx.at[pl.ds(base, NCH)], nidx_v)
    descs = []
    for j in range(NCH):
        rows = pl.ds(j * CHUNK, CHUNK)
        descs.append((
            pltpu.async_copy(utab.at[uidx_v.at[j]], urows.at[rows], sems[j]),
            pltpu.async_copy(itab.at[pidx_v.at[j]], prows.at[rows], sems[j]),
            pltpu.async_copy(itab.at[nidx_v.at[j]], nrows.at[rows], sems[j]),
        ))
    for j in range(NCH):
        for dsc in descs[j]:
            dsc.wait()
        rows = pl.ds(j * CHUNK, CHUNK)
        out = pl.ds(wid * BPW + j * CHUNK, CHUNK)
        pltpu.sync_copy(urows.at[rows], urows_hbm.at[out])
        pltpu.sync_copy(prows.at[rows], prows_hbm.at[out])
        pltpu.sync_copy(nrows.at[rows], nrows_hbm.at[out])


_gather_kernel = functools.partial(
    pl.kernel,
    mesh=_mesh,
    compiler_params=pltpu.CompilerParams(use_tc_tiling_on_sc=False),
    out_type=(jax.ShapeDtypeStruct((BATCH, D), jnp.float32),
              jax.ShapeDtypeStruct((BATCH, D), jnp.float32),
              jax.ShapeDtypeStruct((BATCH, D), jnp.float32)),
    scratch_types=[
        pltpu.VMEM((NCH, CHUNK), jnp.int32),
        pltpu.VMEM((NCH, CHUNK), jnp.int32),
        pltpu.VMEM((NCH, CHUNK), jnp.int32),
        pltpu.VMEM((BPW, D), jnp.float32),
        pltpu.VMEM((BPW, D), jnp.float32),
        pltpu.VMEM((BPW, D), jnp.float32),
        pltpu.SemaphoreType.DMA,
        pltpu.SemaphoreType.DMA,
        pltpu.SemaphoreType.DMA,
        pltpu.SemaphoreType.DMA,
    ],
)(_gather_body)


def _dot_body(u_ref, p_ref, n_ref, pos_ref, neg_ref):
    u = u_ref[...]
    pos_ref[...] = jnp.sum(u * p_ref[...], axis=1)
    neg_ref[...] = jnp.sum(u * n_ref[...], axis=1)


_DOT_ROWS = 2048


def _dot(urows, prows, nrows):
    grid = BATCH // _DOT_ROWS
    return pl.pallas_call(
        _dot_body,
        grid=(grid,),
        in_specs=[pl.BlockSpec((_DOT_ROWS, D), lambda i: (i, 0))] * 3,
        out_specs=[pl.BlockSpec((_DOT_ROWS,), lambda i: (i,))] * 2,
        out_shape=[jax.ShapeDtypeStruct((BATCH,), jnp.float32)] * 2,
    )(urows, prows, nrows)


def kernel(user_indices, pos_item_indices, neg_item_indices, user_table, item_table):
    u2 = user_indices.astype(jnp.int32).reshape(NW * NCH, CHUNK)
    p2 = pos_item_indices.astype(jnp.int32).reshape(NW * NCH, CHUNK)
    n2 = neg_item_indices.astype(jnp.int32).reshape(NW * NCH, CHUNK)
    urows, prows, nrows = _gather_kernel(u2, p2, n2, user_table, item_table)
    pos, neg = _dot(urows, prows, nrows)
    return pos, neg
